# Initial kernel scaffold; baseline (speedup 1.0000x reference)
import jax
import jax.numpy as jnp
from jax.experimental import pallas as pl


def kernel(inputs, embeddings):
    raise NotImplementedError("write your pallas kernel here")


# speedup vs baseline: 3.3031x; 3.3031x over previous
import functools

import numpy as np
import jax
import jax.numpy as jnp
from jax import lax
from jax.experimental import pallas as pl
from jax.experimental.pallas import tpu as pltpu
from jax.experimental.pallas import tpu_sc as plsc

NUM_LEVELS = 16
LEVEL_DIM = 2
P1 = int(np.uint32(2654435761).view(np.int32))
P2 = 805459861
HASH_MASK = (1 << 19) - 1

_OFFS = [0, 4096, 36864, 299008]
for _ in range(13):
    _OFFS.append(_OFFS[-1] + (1 << 19))

NC, NS = 2, 16
NW = NC * NS
CH = 128
NG = CH // 16


def _encode_body(spw, nch, inp, emb0, emb1, out, xyz, idxb, wb, rows0, rows1, outb, gsem):
    cid = lax.axis_index("c")
    sid = lax.axis_index("s")
    wid = sid * NC + cid
    iota = lax.iota(jnp.int32, 16)
    zero16 = jnp.zeros((16,), jnp.int32)
    one16 = jnp.full((16,), 1, jnp.int32)

    def corner_w(wa, wbv, c):
        w = wbv[0] if (c & 1) else wa[0]
        w = w * (wbv[1] if (c >> 1) & 1 else wa[1])
        w = w * (wbv[2] if (c >> 2) & 1 else wa[2])
        return w

    def load_x01(g16):
        return [(xyz[d, pl.ds(g16, 16)] + 1.0) * 0.5 for d in range(3)]

    def smooth(xs, scale_f):
        li, wa, wbv = [], [], []
        for d in range(3):
            pos = xs[d] * scale_f
            lid = pos.astype(jnp.int32)
            t = pos - lid.astype(jnp.float32)
            w = (t * t) * (3.0 - (t + t))
            li.append(lid)
            wbv.append(w)
            wa.append(1.0 - w)
        return li, wa, wbv

    @pl.loop(0, nch)
    def _chunk(ci):
        base = wid * spw + ci * CH
        pltpu.sync_copy(inp.at[:, pl.ds(base, CH)], xyz)

        for lvl in range(3):
            res = 16 << lvl

            @pl.loop(0, NG)
            def _g(g, lvl=lvl, res=res):
                g16 = g * 16
                xs = load_x01(g16)
                li, wa, wbv = smooth(xs, float(res - 1))
                s0 = li[0] + li[1] * res + li[2] * (res * res) + _OFFS[lvl]
                for c in range(8):
                    cc = (c & 1) + ((c >> 1) & 1) * res + ((c >> 2) & 1) * (res * res)
                    row = lvl * 8 + c
                    idxb[row, pl.ds(g16, 16)] = s0 + cc
                    wb[row, pl.ds(g16, 16)] = corner_w(wa, wbv, c)

            for c in range(8):
                row = lvl * 8 + c
                pltpu.async_copy(emb0.at[idxb.at[row]], rows0.at[row], gsem)
                pltpu.async_copy(emb1.at[idxb.at[row]], rows1.at[row], gsem)

        @pl.loop(3, NUM_LEVELS)
        def _lvl(lvl):
            res = 16 << lvl
            scale_f = (res - 1).astype(jnp.float32)
            off = _OFFS[3] + (lvl - 3) * (1 << 19)
            row0 = lvl * 8

            @pl.loop(0, NG)
            def _g(g):
                g16 = g * 16
                xs = load_x01(g16)
                li, wa, wbv = smooth(xs, scale_f)
                hx = (li[0], li[0] + 1)
                hy0 = li[1] * P1
                hy = (hy0, hy0 + P1)
                hz0 = li[2] * P2
                hz = (hz0, hz0 + P2)
                for c in range(8):
                    h = hx[c & 1] ^ hy[(c >> 1) & 1] ^ hz[(c >> 2) & 1]
                    row = row0 + c
                    idxb[row, pl.ds(g16, 16)] = (h & HASH_MASK) + off
                    wb[row, pl.ds(g16, 16)] = corner_w(wa, wbv, c)

            for c in range(8):
                row = row0 + c
                pltpu.async_copy(emb0.at[idxb.at[row]], rows0.at[row], gsem)
                pltpu.async_copy(emb1.at[idxb.at[row]], rows1.at[row], gsem)

        @pl.loop(0, NUM_LEVELS)
        def _acc(lvl):
            row0 = lvl * 8
            for c in range(8):
                row = row0 + c
                pltpu.make_async_copy(emb0.at[idxb.at[row]], rows0.at[row], gsem).wait()
                pltpu.make_async_copy(emb1.at[idxb.at[row]], rows1.at[row], gsem).wait()

            @pl.loop(0, NG)
            def _g(g):
                lane = iota + g * 16
                acc0 = jnp.zeros((16,), jnp.float32)
                acc1 = jnp.zeros((16,), jnp.float32)
                for c in range(8):
                    row = row0 + c
                    wv = wb[row, pl.ds(g * 16, 16)]
                    v0 = rows0[row, pl.ds(g * 16, 16)]
                    v1 = rows1[row, pl.ds(g * 16, 16)]
                    acc0 = acc0 + wv * v0
                    acc1 = acc1 + wv * v1
                outb[lvl * 2, pl.ds(g * 16, 16)] = acc0
                outb[lvl * 2 + 1, pl.ds(g * 16, 16)] = acc1

        pltpu.sync_copy(outb, out.at[:, pl.ds(base, CH)])


def kernel(inputs, embeddings):
    b = inputs.shape[0]
    assert b % (NW * CH) == 0
    spw = b // NW
    nch = spw // CH
    mesh = plsc.VectorSubcoreMesh(
        core_axis_name="c", subcore_axis_name="s", num_cores=NC, num_subcores=NS
    )
    fn = pl.kernel(
        functools.partial(_encode_body, spw, nch),
        out_type=jax.ShapeDtypeStruct((NUM_LEVELS * LEVEL_DIM, b), jnp.float32),
        mesh=mesh,
        scratch_types=[
            pltpu.VMEM((3, CH), jnp.float32),
            pltpu.VMEM((NUM_LEVELS * 8, CH), jnp.int32),
            pltpu.VMEM((NUM_LEVELS * 8, CH), jnp.float32),
            pltpu.VMEM((NUM_LEVELS * 8, CH), jnp.float32),
            pltpu.VMEM((NUM_LEVELS * 8, CH), jnp.float32),
            pltpu.VMEM((NUM_LEVELS * LEVEL_DIM, CH), jnp.float32),
            pltpu.SemaphoreType.DMA,
        ],
    )
    embp = embeddings.T
    return fn(inputs.T, embp[0], embp[1]).T


# speedup vs baseline: 3.3069x; 1.0012x over previous
import functools

import numpy as np
import jax
import jax.numpy as jnp
from jax import lax
from jax.experimental import pallas as pl
from jax.experimental.pallas import tpu as pltpu
from jax.experimental.pallas import tpu_sc as plsc

NUM_LEVELS = 16
LEVEL_DIM = 2
P1 = int(np.uint32(2654435761).view(np.int32))
P2 = 805459861
HASH_MASK = (1 << 19) - 1

_OFFS = [0, 4096, 36864, 299008]
for _ in range(13):
    _OFFS.append(_OFFS[-1] + (1 << 19))

NC, NS = 2, 16
NW = NC * NS
CH = 128
NG = CH // 16


def _encode_body(
    spw, nch, inp, emb0, emb1, out,
    xyzA, xyzB, idxA, idxB, r0A, r1A, r0B, r1B, outb, semA, semB,
):
    cid = lax.axis_index("c")
    sid = lax.axis_index("s")
    wid = sid * NC + cid

    def corner_w(wa, wbv, c):
        w = wbv[0] if (c & 1) else wa[0]
        w = w * (wbv[1] if (c >> 1) & 1 else wa[1])
        w = w * (wbv[2] if (c >> 2) & 1 else wa[2])
        return w

    def smooth(xyz, g16, scale_f):
        li, wa, wbv = [], [], []
        for d in range(3):
            x01 = (xyz[d, pl.ds(g16, 16)] + 1.0) * 0.5
            pos = x01 * scale_f
            lid = pos.astype(jnp.int32)
            t = pos - lid.astype(jnp.float32)
            w = (t * t) * (3.0 - (t + t))
            li.append(lid)
            wbv.append(w)
            wa.append(1.0 - w)
        return li, wa, wbv

    def compute_chunk(ci, xyz, idxb, rows0, rows1, sem):
        base = wid * spw + ci * CH
        pltpu.sync_copy(inp.at[:, pl.ds(base, CH)], xyz)

        for lvl in range(3):
            res = 16 << lvl

            @pl.loop(0, NG)
            def _g(g, lvl=lvl, res=res):
                g16 = g * 16
                li, _, _ = smooth(xyz, g16, float(res - 1))
                s0 = li[0] + li[1] * res + li[2] * (res * res) + _OFFS[lvl]
                for c in range(8):
                    cc = (c & 1) + ((c >> 1) & 1) * res + ((c >> 2) & 1) * (res * res)
                    idxb[lvl * 8 + c, pl.ds(g16, 16)] = s0 + cc

            for c in range(8):
                row = lvl * 8 + c
                pltpu.async_copy(emb0.at[idxb.at[row]], rows0.at[row], sem)
                pltpu.async_copy(emb1.at[idxb.at[row]], rows1.at[row], sem)

        @pl.loop(3, NUM_LEVELS)
        def _lvl(lvl):
            res = 16 << lvl
            scale_f = (res - 1).astype(jnp.float32)
            off = _OFFS[3] + (lvl - 3) * (1 << 19)
            row0 = lvl * 8

            @pl.loop(0, NG)
            def _g(g):
                g16 = g * 16
                li, _, _ = smooth(xyz, g16, scale_f)
                hx = (li[0], li[0] + 1)
                hy0 = li[1] * P1
                hy = (hy0, hy0 + P1)
                hz0 = li[2] * P2
                hz = (hz0, hz0 + P2)
                for c in range(8):
                    h = hx[c & 1] ^ hy[(c >> 1) & 1] ^ hz[(c >> 2) & 1]
                    idxb[row0 + c, pl.ds(g16, 16)] = (h & HASH_MASK) + off

            for c in range(8):
                row = row0 + c
                pltpu.async_copy(emb0.at[idxb.at[row]], rows0.at[row], sem)
                pltpu.async_copy(emb1.at[idxb.at[row]], rows1.at[row], sem)

    def accum_chunk(ci, xyz, idxb, rows0, rows1, sem):
        base = wid * spw + ci * CH

        @pl.loop(0, NUM_LEVELS)
        def _acc(lvl):
            row0 = lvl * 8
            for c in range(8):
                row = row0 + c
                pltpu.make_async_copy(emb0.at[idxb.at[row]], rows0.at[row], sem).wait()
                pltpu.make_async_copy(emb1.at[idxb.at[row]], rows1.at[row], sem).wait()
            res = 16 << lvl
            scale_f = (res - 1).astype(jnp.float32)

            @pl.loop(0, NG)
            def _g(g):
                g16 = g * 16
                _, wa, wbv = smooth(xyz, g16, scale_f)
                acc0 = jnp.zeros((16,), jnp.float32)
                acc1 = jnp.zeros((16,), jnp.float32)
                for c in range(8):
                    row = row0 + c
                    wv = corner_w(wa, wbv, c)
                    acc0 = acc0 + wv * rows0[row, pl.ds(g16, 16)]
                    acc1 = acc1 + wv * rows1[row, pl.ds(g16, 16)]
                outb[lvl * 2, pl.ds(g16, 16)] = acc0
                outb[lvl * 2 + 1, pl.ds(g16, 16)] = acc1

        pltpu.sync_copy(outb, out.at[:, pl.ds(base, CH)])

    compute_chunk(0, xyzA, idxA, r0A, r1A, semA)

    @pl.loop(0, nch // 2)
    def _pair(p):
        even = p * 2
        compute_chunk(even + 1, xyzB, idxB, r0B, r1B, semB)
        accum_chunk(even, xyzA, idxA, r0A, r1A, semA)

        @pl.when(even + 2 < nch)
        def _():
            compute_chunk(even + 2, xyzA, idxA, r0A, r1A, semA)

        accum_chunk(even + 1, xyzB, idxB, r0B, r1B, semB)


def kernel(inputs, embeddings):
    b = inputs.shape[0]
    assert b % (NW * CH * 2) == 0
    spw = b // NW
    nch = spw // CH
    mesh = plsc.VectorSubcoreMesh(
        core_axis_name="c", subcore_axis_name="s", num_cores=NC, num_subcores=NS
    )
    fn = pl.kernel(
        functools.partial(_encode_body, spw, nch),
        out_type=jax.ShapeDtypeStruct((NUM_LEVELS * LEVEL_DIM, b), jnp.float32),
        mesh=mesh,
        scratch_types=[
            pltpu.VMEM((3, CH), jnp.float32),
            pltpu.VMEM((3, CH), jnp.float32),
            pltpu.VMEM((NUM_LEVELS * 8, CH), jnp.int32),
            pltpu.VMEM((NUM_LEVELS * 8, CH), jnp.int32),
            pltpu.VMEM((NUM_LEVELS * 8, CH), jnp.float32),
            pltpu.VMEM((NUM_LEVELS * 8, CH), jnp.float32),
            pltpu.VMEM((NUM_LEVELS * 8, CH), jnp.float32),
            pltpu.VMEM((NUM_LEVELS * 8, CH), jnp.float32),
            pltpu.VMEM((NUM_LEVELS * LEVEL_DIM, CH), jnp.float32),
            pltpu.SemaphoreType.DMA,
            pltpu.SemaphoreType.DMA,
        ],
    )
    embp = embeddings.T
    return fn(inputs.T, embp[0], embp[1]).T
